# baseline (speedup 1.0000x reference)
import functools

import jax
import jax.numpy as jnp
from jax import lax
from jax.experimental import pallas as pl
from jax.experimental.pallas import tpu as pltpu
from jax.experimental.pallas import tpu_sc as plsc

NC = 2
NS = 16
L = 16
NW = NC * NS


def _make_mf_kernel(batch: int, n_factors: int):
  bpw = batch // NW
  mesh = plsc.VectorSubcoreMesh(
      core_axis_name="c", subcore_axis_name="s", num_cores=NC, num_subcores=NS)

  @functools.partial(
      pl.kernel,
      out_type=jax.ShapeDtypeStruct((batch,), jnp.float32),
      mesh=mesh,
      compiler_params=pltpu.CompilerParams(
          needs_layout_passes=False, use_tc_tiling_on_sc=False),
      scratch_types=dict(
          pairs=pltpu.VMEM((bpw * 2,), jnp.int32),
          uidx=pltpu.VMEM((bpw,), jnp.int32),
          iidx=pltpu.VMEM((bpw,), jnp.int32),
          urows=pltpu.VMEM((bpw, n_factors), jnp.float32),
          irows=pltpu.VMEM((bpw, n_factors), jnp.float32),
          outv=pltpu.VMEM((bpw,), jnp.float32),
          sem_u=pltpu.SemaphoreType.DMA,
          sem_i=pltpu.SemaphoreType.DMA,
      ),
  )
  def mf(data_hbm, uf_hbm, if_hbm, out_hbm, *, pairs, uidx, iidx, urows,
         irows, outv, sem_u, sem_i):
    wid = lax.axis_index("s") * NC + lax.axis_index("c")
    base = wid * bpw

    pltpu.sync_copy(data_hbm.at[pl.ds(base * 2, bpw * 2)], pairs)

    def deint(blk, carry):
      even = (blk * L + lax.iota(jnp.int32, L)) * 2
      u = plsc.load_gather(pairs, [even])
      i = plsc.load_gather(pairs, [even + 1])
      uidx[pl.ds(blk * L, L)] = u
      iidx[pl.ds(blk * L, L)] = i
      return carry

    lax.fori_loop(0, bpw // L, deint, 0, unroll=4)

    cu = pltpu.async_copy(uf_hbm.at[uidx], urows, sem_u)
    ci = pltpu.async_copy(if_hbm.at[iidx], irows, sem_i)
    cu.wait()
    ci.wait()

    def dot(blk, carry):
      rows = blk * L + lax.iota(jnp.int32, L)
      acc = jnp.zeros((L,), jnp.float32)
      for c in range(n_factors):
        cols = jnp.full((L,), c, jnp.int32)
        u = plsc.load_gather(urows, [rows, cols])
        v = plsc.load_gather(irows, [rows, cols])
        acc = acc + u * v
      outv[pl.ds(blk * L, L)] = acc
      return carry

    lax.fori_loop(0, bpw // L, dot, 0)

    pltpu.sync_copy(outv, out_hbm.at[pl.ds(base, bpw)])

  return mf


def kernel(data, user_factors, item_factors):
  batch, _ = data.shape
  _, n_factors = user_factors.shape
  mf = _make_mf_kernel(batch, n_factors)
  return mf(data.reshape(-1), user_factors, item_factors)


# speedup vs baseline: 3.4664x; 3.4664x over previous
import functools

import jax
import jax.numpy as jnp
from jax import lax
from jax.experimental import pallas as pl
from jax.experimental.pallas import tpu as pltpu
from jax.experimental.pallas import tpu_sc as plsc

NC = 2
NS = 16
L = 16
NW = NC * NS
SPAN = 128
PP = 8


def _make_mf_kernel(batch: int, n_factors: int, n_rows: int):
  bpw = batch // NW
  n_steps = bpw // PP
  n_full = (n_rows // SPAN) * SPAN
  tail = n_rows - n_full
  max_off = n_full - SPAN
  mesh = plsc.VectorSubcoreMesh(
      core_axis_name="c", subcore_axis_name="s", num_cores=NC, num_subcores=NS)

  @functools.partial(
      pl.kernel,
      out_type=jax.ShapeDtypeStruct((batch,), jnp.float32),
      mesh=mesh,
      compiler_params=pltpu.CompilerParams(needs_layout_passes=False),
      scratch_types=dict(
          pairs_v=pltpu.VMEM((bpw * 2,), jnp.int32),
          wins=pltpu.VMEM((2 * PP, n_factors, SPAN), jnp.float32),
          tails=pltpu.VMEM((2 * tail * n_factors,), jnp.float32),
          outv=pltpu.VMEM((bpw + L,), jnp.float32),
          sem=pltpu.SemaphoreType.DMA,
      ),
  )
  def mf(data_hbm, uft_hbm, ift_hbm, utail_hbm, itail_hbm, out_hbm, *,
         pairs_v, wins, tails, outv, sem):
    wid = lax.axis_index("s") * NC + lax.axis_index("c")
    base = wid * bpw

    pltpu.sync_copy(data_hbm.at[pl.ds(base * 2, bpw * 2)], pairs_v)
    pltpu.sync_copy(utail_hbm, tails.at[pl.ds(0, tail * n_factors)])
    pltpu.sync_copy(itail_hbm, tails.at[pl.ds(tail * n_factors,
                                              tail * n_factors)])

    def step(st, carry):
      ids = pairs_v[pl.ds(st * PP * 2, 2 * PP)]
      for k in range(PP):
        uid = ids[2 * k]
        iid = ids[2 * k + 1]
        uoff = pl.multiple_of(
            jnp.minimum((uid // SPAN) * SPAN, max_off), SPAN)
        ioff = pl.multiple_of(
            jnp.minimum((iid // SPAN) * SPAN, max_off), SPAN)
        pltpu.async_copy(uft_hbm.at[:, pl.ds(uoff, SPAN)], wins.at[k], sem)
        pltpu.async_copy(ift_hbm.at[:, pl.ds(ioff, SPAN)], wins.at[PP + k],
                         sem)
      for k in range(2 * PP):
        pltpu.make_async_copy(
            uft_hbm.at[:, pl.ds(0, SPAN)], wins.at[k], sem).wait()

      lanes = lax.iota(jnp.int32, L)
      slot = lanes % PP
      two = (st * PP + slot) * 2
      uids = plsc.load_gather(pairs_v, [two])
      iids = plsc.load_gather(pairs_v, [two + 1])
      urem = uids % SPAN
      irem = iids % SPAN
      u_tail = uids >= n_full
      i_tail = iids >= n_full
      ut_base = jnp.maximum(uids - n_full, 0) * n_factors
      it_base = jnp.maximum(iids - n_full, 0) * n_factors + tail * n_factors
      acc = jnp.zeros((L,), jnp.float32)
      for c in range(n_factors):
        c_vec = jnp.full((L,), c, jnp.int32)
        u = plsc.load_gather(wins, [slot, c_vec, urem])
        v = plsc.load_gather(wins, [slot + PP, c_vec, irem])
        ut = plsc.load_gather(tails, [ut_base + c])
        vt = plsc.load_gather(tails, [it_base + c])
        u = jnp.where(u_tail, ut, u)
        v = jnp.where(i_tail, vt, v)
        acc = acc + u * v
      plsc.store_compressed(outv.at[pl.ds(st * PP, L)], acc, mask=lanes < PP)
      return carry

    lax.fori_loop(0, n_steps, step, 0)

    pltpu.sync_copy(outv.at[pl.ds(0, bpw)], out_hbm.at[pl.ds(base, bpw)])

  return mf


def kernel(data, user_factors, item_factors):
  batch, _ = data.shape
  n_rows, n_factors = user_factors.shape
  n_full = (n_rows // SPAN) * SPAN
  mf = _make_mf_kernel(batch, n_factors, n_rows)
  return mf(data.reshape(-1), user_factors.T, item_factors.T,
            user_factors[n_full:].reshape(-1),
            item_factors[n_full:].reshape(-1))


# speedup vs baseline: 3.7024x; 1.0681x over previous
import functools

import jax
import jax.numpy as jnp
from jax import lax
from jax.experimental import pallas as pl
from jax.experimental.pallas import tpu as pltpu
from jax.experimental.pallas import tpu_sc as plsc

NC = 2
NS = 16
L = 16
SPAN = 128
CHUNK = 512
CAP = 64


def _make_sweep_kernel(batch: int, n_factors: int, n_rows: int):
  n_full = (n_rows // SPAN) * SPAN
  tail = n_rows - n_full
  max_off = n_full - SPAN
  base_chunks = n_full // CHUNK // NS
  rows_per_tec = base_chunks * CHUNK
  last_extra = n_full // CHUNK - base_chunks * NS
  nch = base_chunks + last_extra + 1
  n_groups = batch // L
  mesh = plsc.VectorSubcoreMesh(
      core_axis_name="c", subcore_axis_name="s", num_cores=NC, num_subcores=NS)

  @functools.partial(
      pl.kernel,
      out_type=(jax.ShapeDtypeStruct((batch, n_factors), jnp.float32),
                jax.ShapeDtypeStruct((batch, n_factors), jnp.float32)),
      mesh=mesh,
      compiler_params=pltpu.CompilerParams(needs_layout_passes=False),
      scratch_types=dict(
          pairs=pltpu.VMEM((2 * batch,), jnp.int32),
          lid=pltpu.VMEM((batch + L,), jnp.int32),
          lpd=pltpu.VMEM((batch + L,), jnp.int32),
          bid=pltpu.VMEM((nch * CAP + L,), jnp.int32),
          bpd=pltpu.VMEM((nch * CAP + L,), jnp.int32),
          counts=pltpu.SMEM((nch,), jnp.int32),
          wins=pltpu.VMEM((2, n_factors, CHUNK), jnp.float32),
          ovwin=pltpu.VMEM((n_factors, SPAN), jnp.float32),
          rowbuf=pltpu.VMEM((CAP, n_factors), jnp.float32),
          tails=pltpu.VMEM((2 * tail * n_factors,), jnp.float32),
          sems=pltpu.SemaphoreType.DMA((2,)),
          sem_w=pltpu.SemaphoreType.DMA,
      ),
  )
  def sweep(data_hbm, uft_hbm, ift_hbm, utail_hbm, itail_hbm, urows_hbm,
            irows_hbm, *, pairs, lid, lpd, bid, bpd, counts, wins, ovwin,
            rowbuf, tails, sems, sem_w):
    c = lax.axis_index("c")
    t = lax.axis_index("s")
    lo = t * rows_per_tec
    is_last = t == NS - 1
    hi = jnp.where(is_last, n_rows, lo + rows_per_tec)
    n_sweep = jnp.where(is_last, base_chunks + last_extra, base_chunks)

    pltpu.sync_copy(data_hbm, pairs)
    pltpu.sync_copy(utail_hbm, tails.at[pl.ds(0, tail * n_factors)])
    pltpu.sync_copy(itail_hbm,
                    tails.at[pl.ds(tail * n_factors, tail * n_factors)])
    lanes = lax.iota(jnp.int32, L)
    lane0 = lanes < 1

    def zero_counts(i, carry):
      counts[i] = 0
      return carry

    lax.fori_loop(0, nch, zero_counts, 0)

    def filt(g, cnt):
      ids = plsc.load_gather(pairs, [(g * L + lanes) * 2 + c])
      m = (ids >= lo) & (ids < hi)
      plsc.store_compressed(lid.at[pl.ds(cnt, L)], ids, mask=m)
      plsc.store_compressed(lpd.at[pl.ds(cnt, L)], g * L + lanes, mask=m)
      return cnt + plsc.all_reduce_population_count(m)[0]

    cnt = lax.fori_loop(0, n_groups, filt, 0)

    def extract_row(win, pre, id_s, r_s):
      r = jnp.full((L,), r_s, jnp.int32)
      g0 = plsc.load_gather(win, pre + [lanes, r])
      g1 = plsc.load_gather(win, pre + [lanes + L, r])
      tb = (jnp.maximum(id_s - n_full, 0) * n_factors
            + c * (tail * n_factors))
      t0 = plsc.load_gather(tails, [tb + lanes])
      t1 = plsc.load_gather(tails, [tb + L + lanes])
      in_tail = jnp.full((L,), id_s >= n_full, jnp.bool_)
      return jnp.where(in_tail, t0, g0), jnp.where(in_tail, t1, g1)

    def put_row(slot, r0, r1):
      s = jnp.full((L,), slot, jnp.int32)
      plsc.store_scatter(rowbuf, [s, lanes], r0)
      plsc.store_scatter(rowbuf, [s, lanes + L], r1)

    def run(tbl, out_hbm):
      def bucketize(h, carry):
        id_s = lid[pl.ds(h, L)][0]
        p_s = lpd[pl.ds(h, L)][0]
        ch = (id_s - lo) // CHUNK
        slot = counts[ch]
        counts[ch] = slot + 1

        @pl.when(slot < CAP)
        def _():
          pos = jnp.full((L,), ch * CAP + slot, jnp.int32)
          plsc.store_scatter(bid, [pos], jnp.full((L,), id_s, jnp.int32),
                             mask=lane0)
          plsc.store_scatter(bpd, [pos], jnp.full((L,), p_s, jnp.int32),
                             mask=lane0)

        @pl.when(slot >= CAP)
        def _():
          off = pl.multiple_of(
              jnp.minimum((id_s // SPAN) * SPAN, max_off), SPAN)
          pltpu.sync_copy(tbl.at[:, pl.ds(off, SPAN)], ovwin)
          r0, r1 = extract_row(ovwin, [], id_s, id_s % SPAN)
          put_row(0, r0, r1)
          pltpu.sync_copy(rowbuf.at[0], out_hbm.at[p_s])

        return carry

      lax.fori_loop(0, cnt, bucketize, 0)

      def fire(ck, buf):
        off = pl.multiple_of(lo + ck * CHUNK, SPAN)
        pltpu.async_copy(tbl.at[:, pl.ds(off, CHUNK)], wins.at[buf],
                         sems.at[buf])

      @pl.when(n_sweep > 0)
      def _():
        fire(0, 0)

      def chunk_step(ck, carry):
        buf = ck % 2

        @pl.when(ck + 1 < n_sweep)
        def _():
          fire(ck + 1, 1 - buf)

        pltpu.make_async_copy(tbl.at[:, pl.ds(0, CHUNK)], wins.at[buf],
                              sems.at[buf]).wait()
        nh = jnp.minimum(counts[ck], CAP)

        def hit(h, carry2):
          id_s = bid[pl.ds(ck * CAP + h, L)][0]
          p_s = bpd[pl.ds(ck * CAP + h, L)][0]
          r0, r1 = extract_row(wins, [jnp.full((L,), buf, jnp.int32)], id_s,
                               id_s - (lo + ck * CHUNK))
          put_row(h, r0, r1)
          pltpu.async_copy(rowbuf.at[h], out_hbm.at[p_s], sem_w)
          return carry2

        lax.fori_loop(0, nh, hit, 0)

        def drain(h, carry2):
          pltpu.make_async_copy(rowbuf.at[0], out_hbm.at[0], sem_w).wait()
          return carry2

        lax.fori_loop(0, nh, drain, 0)
        return carry

      lax.fori_loop(0, n_sweep, chunk_step, 0)

      @pl.when(is_last)
      def _():
        tch = nch - 1
        nh = jnp.minimum(counts[tch], CAP)

        def thit(h, carry2):
          id_s = bid[pl.ds(tch * CAP + h, L)][0]
          p_s = bpd[pl.ds(tch * CAP + h, L)][0]
          r0, r1 = extract_row(ovwin, [], id_s, 0)
          put_row(h, r0, r1)
          pltpu.async_copy(rowbuf.at[h], out_hbm.at[p_s], sem_w)
          return carry2

        lax.fori_loop(0, nh, thit, 0)

        def tdrain(h, carry2):
          pltpu.make_async_copy(rowbuf.at[0], out_hbm.at[0], sem_w).wait()
          return carry2

        lax.fori_loop(0, nh, tdrain, 0)

    @pl.when(c == 0)
    def _():
      run(uft_hbm, urows_hbm)

    @pl.when(c == 1)
    def _():
      run(ift_hbm, irows_hbm)

  return sweep


def _make_dot_kernel(batch: int, n_factors: int, blk: int = 512):
  def body(u_ref, v_ref, o_ref):
    o_ref[...] = jnp.sum(u_ref[...] * v_ref[...], axis=1)

  return pl.pallas_call(
      body,
      grid=(batch // blk,),
      in_specs=[pl.BlockSpec((blk, n_factors), lambda i: (i, 0))] * 2,
      out_specs=pl.BlockSpec((blk,), lambda i: (i,)),
      out_shape=jax.ShapeDtypeStruct((batch,), jnp.float32),
  )


def kernel(data, user_factors, item_factors):
  batch, _ = data.shape
  n_rows, n_factors = user_factors.shape
  n_full = (n_rows // SPAN) * SPAN
  sweep = _make_sweep_kernel(batch, n_factors, n_rows)
  urows, irows = sweep(data.reshape(-1), user_factors.T, item_factors.T,
                       user_factors[n_full:].reshape(-1),
                       item_factors[n_full:].reshape(-1))
  return _make_dot_kernel(batch, n_factors)(urows, irows)


# speedup vs baseline: 4.2076x; 1.1365x over previous
import functools

import jax
import jax.numpy as jnp
from jax import lax
from jax.experimental import pallas as pl
from jax.experimental.pallas import tpu as pltpu
from jax.experimental.pallas import tpu_sc as plsc

NC = 2
NS = 16
L = 16
SPAN = 128
CHUNK = 128
NBUF = 8
CAP = 16


def _make_sweep_kernel(batch: int, n_factors: int, n_rows: int):
  n_full = (n_rows // SPAN) * SPAN
  tail = n_rows - n_full
  max_off = n_full - SPAN
  base_chunks = n_full // CHUNK // NS
  rows_per_tec = base_chunks * CHUNK
  last_extra = n_full // CHUNK - base_chunks * NS
  nch = base_chunks + last_extra + 1
  n_groups = batch // L
  mesh = plsc.VectorSubcoreMesh(
      core_axis_name="c", subcore_axis_name="s", num_cores=NC, num_subcores=NS)

  @functools.partial(
      pl.kernel,
      out_type=(jax.ShapeDtypeStruct((batch, n_factors), jnp.float32),
                jax.ShapeDtypeStruct((batch, n_factors), jnp.float32)),
      mesh=mesh,
      compiler_params=pltpu.CompilerParams(needs_layout_passes=False),
      scratch_types=dict(
          pairs=pltpu.VMEM((2 * batch,), jnp.int32),
          lid=pltpu.VMEM((batch + L,), jnp.int32),
          lpd=pltpu.VMEM((batch + L,), jnp.int32),
          bid=pltpu.VMEM((nch * CAP + L,), jnp.int32),
          bpd=pltpu.VMEM((nch * CAP + L,), jnp.int32),
          counts=pltpu.SMEM((nch,), jnp.int32),
          wins=pltpu.VMEM((NBUF, n_factors, CHUNK), jnp.float32),
          ovwin=pltpu.VMEM((n_factors, SPAN), jnp.float32),
          rowbuf=pltpu.VMEM((CAP, n_factors), jnp.float32),
          tails=pltpu.VMEM((2 * tail * n_factors,), jnp.float32),
          sems=pltpu.SemaphoreType.DMA((NBUF,)),
          sem_w=pltpu.SemaphoreType.DMA,
      ),
  )
  def sweep(data_hbm, uft_hbm, ift_hbm, utail_hbm, itail_hbm, urows_hbm,
            irows_hbm, *, pairs, lid, lpd, bid, bpd, counts, wins, ovwin,
            rowbuf, tails, sems, sem_w):
    c = lax.axis_index("c")
    t = lax.axis_index("s")
    lo = t * rows_per_tec
    is_last = t == NS - 1
    hi = jnp.where(is_last, n_rows, lo + rows_per_tec)
    n_sweep = jnp.where(is_last, base_chunks + last_extra, base_chunks)

    pltpu.sync_copy(data_hbm, pairs)
    pltpu.sync_copy(utail_hbm, tails.at[pl.ds(0, tail * n_factors)])
    pltpu.sync_copy(itail_hbm,
                    tails.at[pl.ds(tail * n_factors, tail * n_factors)])
    lanes = lax.iota(jnp.int32, L)
    lane0 = lanes < 1

    def zero_counts(i, carry):
      counts[i] = 0
      return carry

    lax.fori_loop(0, nch, zero_counts, 0)

    def filt(g, cnt):
      ids = plsc.load_gather(pairs, [(g * L + lanes) * 2 + c])
      m = (ids >= lo) & (ids < hi)
      plsc.store_compressed(lid.at[pl.ds(cnt, L)], ids, mask=m)
      plsc.store_compressed(lpd.at[pl.ds(cnt, L)], g * L + lanes, mask=m)
      return cnt + plsc.all_reduce_population_count(m)[0]

    cnt = lax.fori_loop(0, n_groups, filt, 0)

    def extract_row(win, pre, id_s, r_s):
      r = jnp.full((L,), r_s, jnp.int32)
      g0 = plsc.load_gather(win, pre + [lanes, r])
      g1 = plsc.load_gather(win, pre + [lanes + L, r])
      tb = (jnp.maximum(id_s - n_full, 0) * n_factors
            + c * (tail * n_factors))
      t0 = plsc.load_gather(tails, [tb + lanes])
      t1 = plsc.load_gather(tails, [tb + L + lanes])
      in_tail = jnp.full((L,), id_s >= n_full, jnp.bool_)
      return jnp.where(in_tail, t0, g0), jnp.where(in_tail, t1, g1)

    def put_row(slot, r0, r1):
      s = jnp.full((L,), slot, jnp.int32)
      plsc.store_scatter(rowbuf, [s, lanes], r0)
      plsc.store_scatter(rowbuf, [s, lanes + L], r1)

    def run(tbl, out_hbm):
      def bucketize(h, carry):
        id_s = lid[pl.ds(h, L)][0]
        p_s = lpd[pl.ds(h, L)][0]
        ch = (id_s - lo) // CHUNK
        slot = counts[ch]
        counts[ch] = slot + 1

        @pl.when(slot < CAP)
        def _():
          pos = jnp.full((L,), ch * CAP + slot, jnp.int32)
          plsc.store_scatter(bid, [pos], jnp.full((L,), id_s, jnp.int32),
                             mask=lane0)
          plsc.store_scatter(bpd, [pos], jnp.full((L,), p_s, jnp.int32),
                             mask=lane0)

        @pl.when(slot >= CAP)
        def _():
          off = pl.multiple_of(
              jnp.minimum((id_s // SPAN) * SPAN, max_off), SPAN)
          pltpu.sync_copy(tbl.at[:, pl.ds(off, SPAN)], ovwin)
          r0, r1 = extract_row(ovwin, [], id_s, id_s % SPAN)
          put_row(0, r0, r1)
          pltpu.sync_copy(rowbuf.at[0], out_hbm.at[p_s])

        return carry

      lax.fori_loop(0, cnt, bucketize, 0)

      def fire(ck, buf):
        off = pl.multiple_of(lo + ck * CHUNK, SPAN)
        pltpu.async_copy(tbl.at[:, pl.ds(off, CHUNK)], wins.at[buf],
                         sems.at[buf])

      for j in range(NBUF - 1):

        @pl.when(j < n_sweep)
        def _(j=j):
          fire(j, j)

      def chunk_step(ck, carry):
        buf = ck % NBUF

        @pl.when(ck + NBUF - 1 < n_sweep)
        def _():
          fire(ck + NBUF - 1, (ck + NBUF - 1) % NBUF)

        pltpu.make_async_copy(tbl.at[:, pl.ds(0, CHUNK)], wins.at[buf],
                              sems.at[buf]).wait()
        nh = jnp.minimum(counts[ck], CAP)

        def hit(h, carry2):
          id_s = bid[pl.ds(ck * CAP + h, L)][0]
          p_s = bpd[pl.ds(ck * CAP + h, L)][0]
          r0, r1 = extract_row(wins, [jnp.full((L,), buf, jnp.int32)], id_s,
                               id_s - (lo + ck * CHUNK))
          put_row(h, r0, r1)
          pltpu.async_copy(rowbuf.at[h], out_hbm.at[p_s], sem_w)
          return carry2

        lax.fori_loop(0, nh, hit, 0)

        def drain(h, carry2):
          pltpu.make_async_copy(rowbuf.at[0], out_hbm.at[0], sem_w).wait()
          return carry2

        lax.fori_loop(0, nh, drain, 0)
        return carry

      lax.fori_loop(0, n_sweep, chunk_step, 0)

      @pl.when(is_last)
      def _():
        tch = nch - 1
        nh = jnp.minimum(counts[tch], CAP)

        def thit(h, carry2):
          id_s = bid[pl.ds(tch * CAP + h, L)][0]
          p_s = bpd[pl.ds(tch * CAP + h, L)][0]
          r0, r1 = extract_row(ovwin, [], id_s, 0)
          put_row(h, r0, r1)
          pltpu.async_copy(rowbuf.at[h], out_hbm.at[p_s], sem_w)
          return carry2

        lax.fori_loop(0, nh, thit, 0)

        def tdrain(h, carry2):
          pltpu.make_async_copy(rowbuf.at[0], out_hbm.at[0], sem_w).wait()
          return carry2

        lax.fori_loop(0, nh, tdrain, 0)

    @pl.when(c == 0)
    def _():
      run(uft_hbm, urows_hbm)

    @pl.when(c == 1)
    def _():
      run(ift_hbm, irows_hbm)

  return sweep


def _make_dot_kernel(batch: int, n_factors: int, blk: int = 512):
  def body(u_ref, v_ref, o_ref):
    o_ref[...] = jnp.sum(u_ref[...] * v_ref[...], axis=1)

  return pl.pallas_call(
      body,
      grid=(batch // blk,),
      in_specs=[pl.BlockSpec((blk, n_factors), lambda i: (i, 0))] * 2,
      out_specs=pl.BlockSpec((blk,), lambda i: (i,)),
      out_shape=jax.ShapeDtypeStruct((batch,), jnp.float32),
  )


def kernel(data, user_factors, item_factors):
  batch, _ = data.shape
  n_rows, n_factors = user_factors.shape
  n_full = (n_rows // SPAN) * SPAN
  sweep = _make_sweep_kernel(batch, n_factors, n_rows)
  urows, irows = sweep(data.reshape(-1), user_factors.T, item_factors.T,
                       user_factors[n_full:].reshape(-1),
                       item_factors[n_full:].reshape(-1))
  return _make_dot_kernel(batch, n_factors)(urows, irows)
